# Initial kernel scaffold; baseline (speedup 1.0000x reference)
import jax
import jax.numpy as jnp
from jax.experimental import pallas as pl


def kernel(x, table):
    raise NotImplementedError("write your pallas kernel here")


# speedup vs baseline: 5.7673x; 5.7673x over previous
import functools

import jax
import jax.numpy as jnp
from jax import lax
from jax.experimental import pallas as pl
from jax.experimental.pallas import tpu as pltpu
from jax.experimental.pallas import tpu_sc as plsc

VOCAB = 100000
DIM = 128
BATCH = 4096
SEQ = 200
NWIN = SEQ - 2
LANES = 16
NCHUNK = DIM // LANES

NUM_CORES = 2
NUM_SUBCORES = 16
NW = NUM_CORES * NUM_SUBCORES
BPW = BATCH // NW

SPLIT = 96


def _build():
  mesh = plsc.VectorSubcoreMesh(core_axis_name="c", subcore_axis_name="s")

  @functools.partial(
      pl.kernel,
      out_type=jax.ShapeDtypeStruct((BATCH, DIM), jnp.float32),
      mesh=mesh,
      scratch_types=[
          pltpu.VMEM((BPW * SEQ,), jnp.int32),
          pltpu.VMEM((SEQ, DIM), jnp.float32),
          pltpu.VMEM((BPW, DIM), jnp.float32),
          pltpu.SemaphoreType.DMA,
      ],
      compiler_params=pltpu.CompilerParams(needs_layout_passes=False),
  )
  def encode(x_hbm, table_hbm, out_hbm, x_v, rows_v, out_v, sem):
    wid = lax.axis_index("s") * NUM_CORES + lax.axis_index("c")
    base = pl.multiple_of(wid * BPW, 8)
    pltpu.sync_copy(x_hbm.at[pl.ds(base * SEQ, BPW * SEQ)], x_v)

    iota = lax.iota(jnp.int32, LANES)
    lanes_a = [(iota + (16 * k + DIM - 2)) % DIM for k in range(NCHUNK)]
    lanes_b = [(iota + (16 * k + DIM - 1)) % DIM for k in range(NCHUNK)]
    lanes_c = [iota + 16 * k for k in range(NCHUNK)]

    def body_b(b, carry):
      off = pl.multiple_of(b * SEQ, 8)
      cp1 = pltpu.async_copy(
          table_hbm.at[x_v.at[pl.ds(off, SPLIT)]],
          rows_v.at[pl.ds(0, SPLIT)], sem)
      cp2 = pltpu.async_copy(
          table_hbm.at[x_v.at[pl.ds(off + SPLIT, SEQ - SPLIT)]],
          rows_v.at[pl.ds(SPLIT, SEQ - SPLIT)], sem)
      cp1.wait()
      cp2.wait()

      def body_t(t, accs):
        t_vec = jnp.full((LANES,), t, jnp.int32)
        t1_vec = t_vec + 1
        t2_vec = t_vec + 2
        new = []
        for k in range(NCHUNK):
          a = plsc.load_gather(rows_v, [t_vec, lanes_a[k]])
          bb = plsc.load_gather(rows_v, [t1_vec, lanes_b[k]])
          c = plsc.load_gather(rows_v, [t2_vec, lanes_c[k]])
          new.append(accs[k] + a * bb * c)
        return tuple(new)

      accs = lax.fori_loop(
          0, NWIN, body_t,
          tuple(jnp.zeros((LANES,), jnp.float32) for _ in range(NCHUNK)))
      one = jnp.full((LANES,), 1.0, jnp.float32)
      b_vec = jnp.full((LANES,), b, jnp.int32)
      for k in range(NCHUNK):
        plsc.store_scatter(out_v, [b_vec, lanes_c[k]],
                           jnp.where(accs[k] > 0, one, -one))
      return carry

    lax.fori_loop(0, BPW, body_b, 0)
    pltpu.sync_copy(out_v, out_hbm.at[pl.ds(base, BPW)])

  return encode


_encode = _build()


def kernel(x, table):
  x_flat = x.astype(jnp.int32).reshape(BATCH * SEQ)
  return _encode(x_flat, table)


# speedup vs baseline: 7.9780x; 1.3833x over previous
import functools

import jax
import jax.numpy as jnp
from jax import lax
from jax.experimental import pallas as pl
from jax.experimental.pallas import tpu as pltpu
from jax.experimental.pallas import tpu_sc as plsc

VOCAB = 100000
DIM = 128
BATCH = 4096
SEQ = 200
NWIN = SEQ - 2
LANES = 16
NCHUNK = DIM // LANES

NUM_CORES = 2
NUM_SUBCORES = 16
NW = NUM_CORES * NUM_SUBCORES
BPW = BATCH // NW
HALF = BPW // 2

SPLIT = 96


def _build():
  mesh = plsc.VectorSubcoreMesh(core_axis_name="c", subcore_axis_name="s")

  @functools.partial(
      pl.kernel,
      out_type=jax.ShapeDtypeStruct((BATCH, DIM), jnp.float32),
      mesh=mesh,
      scratch_types=[
          pltpu.VMEM((BPW * SEQ,), jnp.int32),
          pltpu.VMEM((SEQ, DIM), jnp.float32),
          pltpu.VMEM((SEQ, DIM), jnp.float32),
          pltpu.VMEM((BPW, DIM), jnp.float32),
          pltpu.SemaphoreType.DMA,
          pltpu.SemaphoreType.DMA,
      ],
      compiler_params=pltpu.CompilerParams(needs_layout_passes=False),
  )
  def encode(x_hbm, table_hbm, out_hbm, x_v, rows_a, rows_b, out_v,
             sem_a, sem_b):
    wid = lax.axis_index("s") * NUM_CORES + lax.axis_index("c")
    base = pl.multiple_of(wid * BPW, 8)
    pltpu.sync_copy(x_hbm.at[pl.ds(base * SEQ, BPW * SEQ)], x_v)

    iota = lax.iota(jnp.int32, LANES)
    lanes_a = [(iota + (16 * k + DIM - 2)) % DIM for k in range(NCHUNK)]
    lanes_b = [(iota + (16 * k + DIM - 1)) % DIM for k in range(NCHUNK)]
    lanes_c = [iota + 16 * k for k in range(NCHUNK)]

    def issue(b, buf, sem):
      off = pl.multiple_of(b * SEQ, 8)
      pltpu.async_copy(
          table_hbm.at[x_v.at[pl.ds(off, SPLIT)]],
          buf.at[pl.ds(0, SPLIT)], sem)
      pltpu.async_copy(
          table_hbm.at[x_v.at[pl.ds(off + SPLIT, SEQ - SPLIT)]],
          buf.at[pl.ds(SPLIT, SEQ - SPLIT)], sem)

    def wait(buf, sem):
      pltpu.make_async_copy(table_hbm.at[pl.ds(0, SEQ)], buf, sem).wait()

    def compute(b, rows_v):
      def body_t(t, accs):
        t_vec = jnp.full((LANES,), t, jnp.int32)
        t1_vec = t_vec + 1
        t2_vec = t_vec + 2
        new = []
        for k in range(NCHUNK):
          a = plsc.load_gather(rows_v, [t_vec, lanes_a[k]])
          bb = plsc.load_gather(rows_v, [t1_vec, lanes_b[k]])
          c = plsc.load_gather(rows_v, [t2_vec, lanes_c[k]])
          new.append(accs[k] + a * bb * c)
        return tuple(new)

      accs = lax.fori_loop(
          0, NWIN, body_t,
          tuple(jnp.zeros((LANES,), jnp.float32) for _ in range(NCHUNK)))
      one = jnp.full((LANES,), 1.0, jnp.float32)
      b_vec = jnp.full((LANES,), b, jnp.int32)
      for k in range(NCHUNK):
        plsc.store_scatter(out_v, [b_vec, lanes_c[k]],
                           jnp.where(accs[k] > 0, one, -one))

    issue(0, rows_a, sem_a)

    def body_pair(i, carry):
      b0 = 2 * i
      issue(b0 + 1, rows_b, sem_b)
      wait(rows_a, sem_a)
      compute(b0, rows_a)

      @pl.when(i < HALF - 1)
      def _():
        issue(b0 + 2, rows_a, sem_a)

      wait(rows_b, sem_b)
      compute(b0 + 1, rows_b)
      return carry

    lax.fori_loop(0, HALF, body_pair, 0)
    pltpu.sync_copy(out_v, out_hbm.at[pl.ds(base, BPW)])

  return encode


_encode = _build()


def kernel(x, table):
  x_flat = x.astype(jnp.int32).reshape(BATCH * SEQ)
  return _encode(x_flat, table)


# speedup vs baseline: 24.2234x; 3.0363x over previous
import functools

import jax
import jax.numpy as jnp
from jax import lax
from jax.experimental import pallas as pl
from jax.experimental.pallas import tpu as pltpu
from jax.experimental.pallas import tpu_sc as plsc

VOCAB = 100000
DIM = 128
BATCH = 4096
SEQ = 200
NWIN = SEQ - 2
LANES = 16
NWORD = DIM // 32
NPLANE = 8
NBND = VOCAB // LANES

NTILE = 32
RPT = BATCH // (NTILE // NWORD)
NGRP = RPT // LANES
GSZ = LANES * SEQ

BCHUNK = 4000
NBCHUNK = VOCAB // BCHUNK

PBLK = 2560


def _pack_block(tbl_ref, out_ref):
  t = tbl_ref[...]
  bits = jnp.where(t < 0.0, 1.0, 0.0)
  m_io = lax.broadcasted_iota(jnp.int32, (2 * NWORD, DIM), 0)
  d_io = lax.broadcasted_iota(jnp.int32, (2 * NWORD, DIM), 1)
  bamt = d_io & 31
  is_hi = m_io >= NWORD
  sel = ((d_io >> 5) == (m_io & 3)) & ((bamt >= 16) == is_hi)
  pw = jnp.left_shift(jnp.ones_like(bamt), bamt & 15)
  w = jnp.where(sel, pw, 0).astype(jnp.float32)
  h = lax.dot_general(w, bits, (((1,), (1,)), ((), ())),
                      preferred_element_type=jnp.float32)
  lo = h[0:NWORD, :].astype(jnp.int32)
  hi = h[NWORD:, :].astype(jnp.int32)
  out_ref[...] = lo | (hi << 16)


_pack = pl.pallas_call(
    _pack_block,
    grid=((VOCAB + PBLK - 1) // PBLK,),
    in_specs=[pl.BlockSpec((PBLK, DIM), lambda i: (i, 0))],
    out_specs=pl.BlockSpec((NWORD, PBLK), lambda i: (0, i)),
    out_shape=jax.ShapeDtypeStruct((NWORD, VOCAB), jnp.int32),
)


def _build_encode():
  mesh = plsc.VectorSubcoreMesh(core_axis_name="c", subcore_axis_name="s")

  @functools.partial(
      pl.kernel,
      out_type=jax.ShapeDtypeStruct((DIM, BATCH), jnp.float32),
      mesh=mesh,
      scratch_types=[
          pltpu.VMEM((VOCAB,), jnp.int32),
          pltpu.VMEM((NBND,), jnp.int32),
          pltpu.VMEM((BCHUNK,), jnp.int32),
          pltpu.VMEM((32, RPT), jnp.float32),
          pltpu.SemaphoreType.DMA,
      ],
      compiler_params=pltpu.CompilerParams(needs_layout_passes=False),
  )
  def encode(x_hbm, packed_hbm, out_hbm, plane_v, bnd_v, x_v, out_v, sem):
    wid = lax.axis_index("s") * 2 + lax.axis_index("c")
    tg = wid // NWORD
    jj = wid % NWORD
    rowbase = pl.multiple_of(tg * RPT, 8)
    jm1 = (jj - 1) % NWORD

    pltpu.sync_copy(
        packed_hbm.at[pl.ds(pl.multiple_of(jj * VOCAB, 8), VOCAB)], plane_v)

    ln = lax.iota(jnp.int32, LANES)
    zero_i = jnp.zeros((LANES,), jnp.int32)
    one_i = jnp.full((LANES,), 1, jnp.int32)
    one_f = jnp.full((LANES,), 1.0, jnp.float32)
    c1v = jnp.full((LANES,), 1, jnp.int32)
    c2v = jnp.full((LANES,), 2, jnp.int32)
    c4v = jnp.full((LANES,), 4, jnp.int32)
    c15v = jnp.full((LANES,), 15, jnp.int32)
    c30v = jnp.full((LANES,), 30, jnp.int32)
    c3v = jnp.full((LANES,), 3, jnp.int32)
    lane_row_off = ln * SEQ

    def bnd_chunk2(c, carry):
      coff = pl.multiple_of(jm1 * VOCAB + c * BCHUNK, 8)
      pltpu.sync_copy(packed_hbm.at[pl.ds(coff, BCHUNK)], x_v)
      nwords = BCHUNK // LANES

      def bnd16(i, carry2):
        w0 = i * LANES
        w0 = jnp.minimum(w0, nwords - LANES)
        wv = jnp.full((LANES,), w0, jnp.int32) + ln
        acc = zero_i
        for k in range(LANES):
          rows = wv * LANES + k
          top2 = lax.shift_right_logical(
              plsc.load_gather(x_v, [rows]), c30v) & c3v
          acc = acc | lax.shift_left(top2, jnp.full((LANES,), 2 * k, jnp.int32))
        plsc.store_scatter(bnd_v, [jnp.full((LANES,), c * nwords, jnp.int32)
                                   + wv], acc)
        return carry2

      lax.fori_loop(0, (nwords + LANES - 1) // LANES, bnd16, 0)
      return carry

    lax.fori_loop(0, NBCHUNK, bnd_chunk2, 0)

    def window(fast, V, slide, fresh_t):
      xt, xt1, wt, wt1, pt, pt1 = slide
      xidx = jnp.full((LANES,), fresh_t, jnp.int32) + lane_row_off
      xt2 = plsc.load_gather(x_v, [xidx])
      wt2 = plsc.load_gather(plane_v, [xt2])
      bw = plsc.load_gather(bnd_v, [lax.shift_right_logical(xt2, c4v)])
      pt2 = lax.shift_right_logical(
          bw, lax.shift_left(xt2 & c15v, c1v)) & c3v
      valid = (xt != 0) & (xt1 != 0) & (xt2 != 0)
      r2 = lax.shift_left(wt, c2v) | pt
      r1 = lax.shift_left(wt1, c1v) | lax.shift_right_logical(pt1, c1v)
      bm = jnp.where(valid, r2 ^ r1 ^ wt2, zero_i)
      a0, a1, a2 = fast
      cr = a0 & bm
      a0 = a0 ^ bm
      cr2 = a1 & cr
      a1 = a1 ^ cr
      a2 = a2 ^ cr2
      V = V + jnp.where(valid, one_i, zero_i)
      return (a0, a1, a2), V, (xt1, xt2, wt1, wt2, pt1, pt2)

    def flush(planes, fast):
      a0, a1, a2 = fast
      c = list(planes)
      cr = c[0] & a0
      c[0] = c[0] ^ a0
      x1 = c[1] ^ a1
      ncr = (c[1] & a1) | (x1 & cr)
      c[1] = x1 ^ cr
      cr = ncr
      x2 = c[2] ^ a2
      ncr = (c[2] & a2) | (x2 & cr)
      c[2] = x2 ^ cr
      cr = ncr
      for k in range(3, NPLANE):
        nk = c[k] ^ cr
        cr = c[k] & cr
        c[k] = nk
      return tuple(c)

    def load_row(t):
      xi = plsc.load_gather(x_v, [jnp.full((LANES,), t, jnp.int32)
                                  + lane_row_off])
      wi = plsc.load_gather(plane_v, [xi])
      bw = plsc.load_gather(bnd_v, [lax.shift_right_logical(xi, c4v)])
      pi = lax.shift_right_logical(bw, lax.shift_left(xi & c15v, c1v)) & c3v
      return xi, wi, pi

    def run_group(g, carry):
      goff = pl.multiple_of((rowbase + g * LANES) * SEQ, 8)
      pltpu.sync_copy(x_hbm.at[pl.ds(goff, GSZ)], x_v.at[pl.ds(0, GSZ)])

      x0, w0, p0 = load_row(0)
      x1, w1, p1 = load_row(1)
      slide0 = (x0, x1, w0, w1, p0, p1)

      def block7(bi, state):
        planes, V, slide = state
        fast = (zero_i, zero_i, zero_i)
        t0 = bi * 7
        for u in range(7):
          fast, V, slide = window(fast, V, slide, t0 + u + 2)
        return flush(planes, fast), V, slide

      init_planes = tuple(zero_i for _ in range(NPLANE))
      planes, V, slide = lax.fori_loop(
          0, NWIN // 7, block7, (init_planes, zero_i, slide0))

      fast = (zero_i, zero_i, zero_i)
      for t in range(NWIN - (NWIN % 7), NWIN):
        fast, V, slide = window(fast, V, slide, t + 2)
      planes = flush(planes, fast)

      cols = jnp.full((LANES,), g * LANES, jnp.int32) + ln

      def unpack_b(b, carry2):
        bv = jnp.full((LANES,), b, jnp.int32)
        cnt = lax.shift_right_logical(planes[0], bv) & one_i
        for k in range(1, NPLANE):
          bit = lax.shift_right_logical(planes[k], bv) & one_i
          cnt = cnt | lax.shift_left(bit, jnp.full((LANES,), k, jnp.int32))
        val = jnp.where(cnt + cnt < V, one_f, -one_f)
        plsc.store_scatter(out_v, [bv, cols], val)
        return carry2

      lax.fori_loop(0, 32, unpack_b, 0)
      return carry

    lax.fori_loop(0, NGRP, run_group, 0)
    pltpu.sync_copy(out_v,
                    out_hbm.at[pl.ds(pl.multiple_of(jj * 32, 8), 32),
                               pl.ds(rowbase, RPT)])

  return encode


_encode = _build_encode()


def kernel(x, table):
  x_flat = x.astype(jnp.int32).reshape(BATCH * SEQ)
  packed = _pack(table).reshape(NWORD * VOCAB)
  out_t = _encode(x_flat, packed)
  return out_t.T


# speedup vs baseline: 34.2183x; 1.4126x over previous
import functools

import jax
import jax.numpy as jnp
from jax import lax
from jax.experimental import pallas as pl
from jax.experimental.pallas import tpu as pltpu
from jax.experimental.pallas import tpu_sc as plsc

VOCAB = 100000
DIM = 128
BATCH = 4096
SEQ = 200
NWIN = SEQ - 2
LANES = 16
NWORD = DIM // 32
NPLANE = 8
NBND = 6272
NPER = NBND // 4

NUM_CORES = 2
NUM_SUBCORES = 16
RPT = BATCH // 8
NGRP = RPT // LANES
NPAIR = NGRP // 2
GSZ = LANES * SEQ

PBLK = 2560


def _pack_block(tbl_ref, out_ref):
  t = tbl_ref[...]
  bits = jnp.where(t < 0.0, 1.0, 0.0)
  m_io = lax.broadcasted_iota(jnp.int32, (2 * NWORD, DIM), 0)
  d_io = lax.broadcasted_iota(jnp.int32, (2 * NWORD, DIM), 1)
  bamt = d_io & 31
  is_hi = m_io >= NWORD
  sel = ((d_io >> 5) == (m_io & 3)) & ((bamt >= 16) == is_hi)
  pw = jnp.left_shift(jnp.ones_like(bamt), bamt & 15)
  w = jnp.where(sel, pw, 0).astype(jnp.float32)
  h = lax.dot_general(w, bits, (((1,), (1,)), ((), ())),
                      preferred_element_type=jnp.float32)
  lo = h[0:NWORD, :].astype(jnp.int32)
  hi = h[NWORD:, :].astype(jnp.int32)
  out_ref[...] = lo | (hi << 16)


_pack = pl.pallas_call(
    _pack_block,
    grid=((VOCAB + PBLK - 1) // PBLK,),
    in_specs=[pl.BlockSpec((PBLK, DIM), lambda i: (i, 0))],
    out_specs=pl.BlockSpec((NWORD, PBLK), lambda i: (0, i)),
    out_shape=jax.ShapeDtypeStruct((NWORD, VOCAB), jnp.int32),
)


def _build_encode():
  mesh = plsc.VectorSubcoreMesh(core_axis_name="c", subcore_axis_name="s")

  @functools.partial(
      pl.kernel,
      out_type=jax.ShapeDtypeStruct((DIM, BATCH), jnp.float32),
      mesh=mesh,
      scratch_types=[
          pltpu.VMEM((VOCAB,), jnp.int32),
          pltpu.VMEM((NBND,), jnp.int32),
          pltpu.VMEM((2 * GSZ,), jnp.int32),
          pltpu.VMEM((32, RPT), jnp.float32),
          pltpu.MemorySpace.VMEM_SHARED((NWORD * NBND,), jnp.int32),
          pltpu.SemaphoreType.DMA,
          pltpu.SemaphoreType.DMA,
      ],
      compiler_params=pltpu.CompilerParams(needs_layout_passes=False),
  )
  def encode(x_hbm, packed_hbm, out_hbm, plane_v, bnd_v, x_v, out_v,
             sbnd, sem_a, sem_b):
    cid = lax.axis_index("c")
    sid = lax.axis_index("s")
    jj = sid % NWORD
    sg = sid // NWORD
    tg = cid * 4 + sg
    rowbase = pl.multiple_of(tg * RPT, 8)
    jp1 = (jj + 1) % NWORD

    pltpu.sync_copy(
        packed_hbm.at[pl.ds(pl.multiple_of(jj * VOCAB, 8), VOCAB)], plane_v)

    ln = lax.iota(jnp.int32, LANES)
    zero_i = jnp.zeros((LANES,), jnp.int32)
    one_i = jnp.full((LANES,), 1, jnp.int32)
    one_f = jnp.full((LANES,), 1.0, jnp.float32)
    c1v = jnp.full((LANES,), 1, jnp.int32)
    c2v = jnp.full((LANES,), 2, jnp.int32)
    c4v = jnp.full((LANES,), 4, jnp.int32)
    c15v = jnp.full((LANES,), 15, jnp.int32)
    c30v = jnp.full((LANES,), 30, jnp.int32)
    c3v = jnp.full((LANES,), 3, jnp.int32)
    lane_row_off = ln * SEQ

    w_lo = pl.multiple_of(sg * NPER, 8)
    vmax = jnp.full((LANES,), VOCAB - 1, jnp.int32)

    def bnd16(i, carry):
      wv = jnp.full((LANES,), w_lo, jnp.int32) + i * LANES + ln
      acc = zero_i
      for k in range(LANES):
        rows = jnp.minimum(wv * LANES + k, vmax)
        top2 = lax.shift_right_logical(
            plsc.load_gather(plane_v, [rows]), c30v) & c3v
        acc = acc | lax.shift_left(top2, jnp.full((LANES,), 2 * k, jnp.int32))
      plsc.store_scatter(bnd_v, [wv], acc)
      return carry

    lax.fori_loop(0, NPER // LANES, bnd16, 0)
    pltpu.sync_copy(
        bnd_v.at[pl.ds(w_lo, NPER)],
        sbnd.at[pl.ds(pl.multiple_of(jp1 * NBND + w_lo, 8), NPER)])
    plsc.subcore_barrier()
    pltpu.sync_copy(
        sbnd.at[pl.ds(pl.multiple_of(jj * NBND, 8), NBND)], bnd_v)

    def compute_group(g, halfoff):
      def window(fast, V, slide, fresh_t):
        f0, f1, wt, wt1, pt, pt1 = slide
        xidx = jnp.full((LANES,), halfoff + fresh_t, jnp.int32) + lane_row_off
        xt2 = plsc.load_gather(x_v, [xidx])
        wt2 = plsc.load_gather(plane_v, [xt2])
        bw = plsc.load_gather(bnd_v, [lax.shift_right_logical(xt2, c4v)])
        pt2 = lax.shift_right_logical(
            bw, lax.shift_left(xt2 & c15v, c1v)) & c3v
        f2 = xt2 != 0
        valid = f0 & f1 & f2
        r2 = lax.shift_left(wt, c2v) | pt
        r1 = lax.shift_left(wt1, c1v) | lax.shift_right_logical(pt1, c1v)
        bm = jnp.where(valid, r2 ^ r1 ^ wt2, zero_i)
        a0, a1, a2 = fast
        cr = a0 & bm
        a0 = a0 ^ bm
        cr2 = a1 & cr
        a1 = a1 ^ cr
        a2 = a2 ^ cr2
        V = V + jnp.where(valid, one_i, zero_i)
        return (a0, a1, a2), V, (f1, f2, wt1, wt2, pt1, pt2)

      def flush(planes, fast):
        a0, a1, a2 = fast
        c = list(planes)
        cr = c[0] & a0
        c[0] = c[0] ^ a0
        x1 = c[1] ^ a1
        ncr = (c[1] & a1) | (x1 & cr)
        c[1] = x1 ^ cr
        cr = ncr
        x2 = c[2] ^ a2
        ncr = (c[2] & a2) | (x2 & cr)
        c[2] = x2 ^ cr
        cr = ncr
        for k in range(3, NPLANE):
          nk = c[k] ^ cr
          cr = c[k] & cr
          c[k] = nk
        return tuple(c)

      def load_row(t):
        xi = plsc.load_gather(
            x_v, [jnp.full((LANES,), halfoff + t, jnp.int32) + lane_row_off])
        wi = plsc.load_gather(plane_v, [xi])
        bw = plsc.load_gather(bnd_v, [lax.shift_right_logical(xi, c4v)])
        pi = lax.shift_right_logical(bw, lax.shift_left(xi & c15v, c1v)) & c3v
        return xi != 0, wi, pi

      f0, w0, p0 = load_row(0)
      f1, w1, p1 = load_row(1)
      slide0 = (f0, f1, w0, w1, p0, p1)

      def block7(bi, state):
        planes, V, slide = state
        fast = (zero_i, zero_i, zero_i)
        t0 = bi * 7
        for u in range(7):
          fast, V, slide = window(fast, V, slide, t0 + u + 2)
        return flush(planes, fast), V, slide

      init_planes = tuple(zero_i for _ in range(NPLANE))
      planes, V, slide = lax.fori_loop(
          0, NWIN // 7, block7, (init_planes, zero_i, slide0))

      fast = (zero_i, zero_i, zero_i)
      for t in range(NWIN - (NWIN % 7), NWIN):
        fast, V, slide = window(fast, V, slide, t + 2)
      planes = flush(planes, fast)

      cols = jnp.full((LANES,), g * LANES, jnp.int32) + ln

      def unpack_b(b, carry2):
        bv = jnp.full((LANES,), b, jnp.int32)
        cnt = lax.shift_right_logical(planes[0], bv) & one_i
        for k in range(1, NPLANE):
          bit = lax.shift_right_logical(planes[k], bv) & one_i
          cnt = cnt | lax.shift_left(bit, jnp.full((LANES,), k, jnp.int32))
        val = jnp.where(cnt + cnt < V, one_f, -one_f)
        plsc.store_scatter(out_v, [bv, cols], val)
        return carry2

      lax.fori_loop(0, 32, unpack_b, 0)

    def issue_x(g, half, sem):
      goff = pl.multiple_of((rowbase + g * LANES) * SEQ, 8)
      pltpu.async_copy(x_hbm.at[pl.ds(goff, GSZ)],
                       x_v.at[pl.ds(half * GSZ, GSZ)], sem)

    def wait_x(half, sem):
      pltpu.make_async_copy(x_hbm.at[pl.ds(0, GSZ)],
                            x_v.at[pl.ds(half * GSZ, GSZ)], sem).wait()

    issue_x(0, 0, sem_a)

    def pair(i, carry):
      g0 = 2 * i
      issue_x(g0 + 1, 1, sem_b)
      wait_x(0, sem_a)
      compute_group(g0, 0)

      @pl.when(i < NPAIR - 1)
      def _():
        issue_x(g0 + 2, 0, sem_a)

      wait_x(1, sem_b)
      compute_group(g0 + 1, GSZ)
      return carry

    lax.fori_loop(0, NPAIR, pair, 0)

    pltpu.sync_copy(out_v,
                    out_hbm.at[pl.ds(pl.multiple_of(jj * 32, 8), 32),
                               pl.ds(rowbase, RPT)])

  return encode


_encode = _build_encode()


def kernel(x, table):
  x_flat = x.astype(jnp.int32).reshape(BATCH * SEQ)
  packed = _pack(table).reshape(NWORD * VOCAB)
  out_t = _encode(x_flat, packed)
  return out_t.T
